# baseline (device time: 14062 ns/iter reference)
import jax
import jax.numpy as jnp
from jax import lax
from jax.experimental import pallas as pl
from jax.experimental.pallas import tpu as pltpu

N_DEV = 8
NSPLIT = 2


def kernel(x):
    m_per, n = x.shape
    m_half = m_per // NSPLIT

    def body(x_ref, out_ref, send_sems, recv_sems):
        my_pos = lax.axis_index("i")

        barrier_sem = pltpu.get_barrier_semaphore()
        for j in range(1, N_DEV):
            pl.semaphore_signal(
                barrier_sem, inc=1,
                device_id=((my_pos + j) % N_DEV,),
                device_id_type=pl.DeviceIdType.MESH,
            )
        pl.semaphore_wait(barrier_sem, N_DEV - 1)

        out_ref[pl.ds(my_pos * m_per, m_per), :] = x_ref[:, :].astype(
            jnp.bfloat16
        )

        sends = []
        for j in range(1, N_DEV):
            for h in range(NSPLIT):
                half = out_ref.at[pl.ds(my_pos * m_per + h * m_half, m_half), :]
                rdma = pltpu.make_async_remote_copy(
                    src_ref=half,
                    dst_ref=half,
                    send_sem=send_sems.at[j - 1, h],
                    recv_sem=recv_sems.at[j - 1, h],
                    device_id=((my_pos + j) % N_DEV,),
                    device_id_type=pl.DeviceIdType.MESH,
                )
                rdma.start()
                sends.append(rdma)

        for k in range(1, N_DEV):
            origin = (my_pos - k) % N_DEV
            for h in range(NSPLIT):
                half = out_ref.at[pl.ds(origin * m_per + h * m_half, m_half), :]
                recv = pltpu.make_async_remote_copy(
                    src_ref=half,
                    dst_ref=half,
                    send_sem=send_sems.at[k - 1, h],
                    recv_sem=recv_sems.at[k - 1, h],
                    device_id=(origin,),
                    device_id_type=pl.DeviceIdType.MESH,
                )
                recv.wait_recv()

        for rdma in sends:
            rdma.wait_send()

    return pl.pallas_call(
        body,
        out_shape=jax.ShapeDtypeStruct((N_DEV * m_per, n), jnp.bfloat16),
        in_specs=[pl.BlockSpec(memory_space=pltpu.VMEM)],
        out_specs=pl.BlockSpec(memory_space=pltpu.VMEM),
        scratch_shapes=[
            pltpu.SemaphoreType.DMA((N_DEV - 1, NSPLIT)),
            pltpu.SemaphoreType.DMA((N_DEV - 1, NSPLIT)),
        ],
        compiler_params=pltpu.CompilerParams(collective_id=0),
    )(x)


# device time: 13168 ns/iter; 1.0679x vs baseline; 1.0679x over previous
import jax
import jax.numpy as jnp
from jax import lax
from jax.experimental import pallas as pl
from jax.experimental.pallas import tpu as pltpu

N_DEV = 8


def kernel(x):
    m_per, n = x.shape

    def body(x_ref, out_ref, send_sems, recv_sems, ready_sems):
        my_pos = lax.axis_index("i")

        barrier_sem = pltpu.get_barrier_semaphore()
        pl.semaphore_signal(
            barrier_sem, inc=1,
            device_id=(my_pos,), device_id_type=pl.DeviceIdType.MESH,
        )
        pl.semaphore_wait(barrier_sem, 1)

        for j in range(1, N_DEV):
            pl.semaphore_signal(
                ready_sems.at[N_DEV - 1 - j], inc=1,
                device_id=((my_pos + j) % N_DEV,),
                device_id_type=pl.DeviceIdType.MESH,
            )

        out_ref[pl.ds(my_pos * m_per, m_per), :] = x_ref[:, :].astype(
            jnp.bfloat16
        )

        my_rows = out_ref.at[pl.ds(my_pos * m_per, m_per), :]
        sends = []
        for j in range(1, N_DEV):
            pl.semaphore_wait(ready_sems.at[j - 1], 1)
            rdma = pltpu.make_async_remote_copy(
                src_ref=my_rows,
                dst_ref=my_rows,
                send_sem=send_sems.at[j - 1],
                recv_sem=recv_sems.at[j - 1],
                device_id=((my_pos + j) % N_DEV,),
                device_id_type=pl.DeviceIdType.MESH,
            )
            rdma.start()
            sends.append(rdma)

        for k in range(1, N_DEV):
            origin = (my_pos - k) % N_DEV
            recv = pltpu.make_async_remote_copy(
                src_ref=my_rows,
                dst_ref=out_ref.at[pl.ds(origin * m_per, m_per), :],
                send_sem=send_sems.at[k - 1],
                recv_sem=recv_sems.at[k - 1],
                device_id=(origin,),
                device_id_type=pl.DeviceIdType.MESH,
            )
            recv.wait_recv()

        for rdma in sends:
            rdma.wait_send()

    return pl.pallas_call(
        body,
        out_shape=jax.ShapeDtypeStruct((N_DEV * m_per, n), jnp.bfloat16),
        in_specs=[pl.BlockSpec(memory_space=pltpu.VMEM)],
        out_specs=pl.BlockSpec(memory_space=pltpu.VMEM),
        scratch_shapes=[
            pltpu.SemaphoreType.DMA((N_DEV - 1,)),
            pltpu.SemaphoreType.DMA((N_DEV - 1,)),
            pltpu.SemaphoreType.REGULAR((N_DEV - 1,)),
        ],
        compiler_params=pltpu.CompilerParams(collective_id=0),
    )(x)
